# baseline (speedup 1.0000x reference)
import jax
import jax.numpy as jnp
from jax import lax
from jax.experimental import pallas as pl
from jax.experimental.pallas import tpu as pltpu
from jax.experimental.pallas import tpu_sc as plsc

N = 10000
E = 320000
D = 128
K = 2
NC = 2
NS = 16
NW = NC * NS
EW = E // NW
CH = 80
NCHUNK = EW // CH
NG = 5
GC = NCHUNK // NG
N2 = 10240
RPT = N2 // NS
ZR = 32
VL = 16
BN = 2000

_mesh = plsc.VectorSubcoreMesh(core_axis_name="c", subcore_axis_name="s")


def _make_sc_layer(DD, tc_tiling=True):
    def body(m_hbm, src_hbm, dst_hbm, attr_hbm, out_hbm,
             src_v, dst_v, attr_v, r0, r1, r2, agg_sh,
             g0, g1, g2, s0, s1, s2):
        cid = lax.axis_index("c")
        sid = lax.axis_index("s")
        wid = sid * NC + cid
        R = (r0, r1, r2)
        G = (g0, g1, g2)
        S = (s0, s1, s2)

        zv = jnp.zeros((VL,), jnp.float32)

        def zb(i, carry):
            r = i // (DD // VL)
            c = (i % (DD // VL)) * VL
            r0[r, pl.ds(c, VL)] = zv
            return carry

        lax.fori_loop(0, CH * (DD // VL), zb, 0)

        def zslab(t, carry):
            pltpu.sync_copy(r0, agg_sh.at[pl.ds(sid * RPT + t * CH, CH)])
            return carry

        lax.fori_loop(0, RPT // CH, zslab, 0)
        plsc.subcore_barrier()

        def gather(i, b):
            pltpu.async_copy(m_hbm.at[src_v.at[i]], R[b], G[b])

        def wait_g(i, b):
            pltpu.make_async_copy(m_hbm.at[src_v.at[i]], R[b], G[b]).wait()

        def scatter(i, b):
            pltpu.async_copy(R[b], agg_sh.at[dst_v.at[i]], S[b], add=True)

        def wait_s(i, b):
            pltpu.make_async_copy(R[b], agg_sh.at[dst_v.at[i]], S[b]).wait()

        def scale(rows_v, i):
            def vgrp(v, c2):
                base = v * VL
                a16 = attr_v[i, pl.ds(base, VL)]
                for jj in range(VL):
                    a = a16[jj]
                    for k in range(DD // VL):
                        sl = pl.ds(k * VL, VL)
                        rows_v[base + jj, sl] = rows_v[base + jj, sl] * a
                return c2

            lax.fori_loop(0, CH // VL, vgrp, 0)

        def step(i, b):
            pb = (b + 2) % 3
            wait_g(i, b)
            scale(R[b], i)
            wait_s(i, pb)

            @pl.when(i + 2 < GC)
            def _():
                gather(i + 2, pb)

            scatter(i, b)

        def group(g, carry):
            pltpu.sync_copy(src_hbm.at[wid, g], src_v)
            pltpu.sync_copy(dst_hbm.at[wid, g], dst_v)
            pltpu.sync_copy(attr_hbm.at[wid, g], attr_v)

            gather(0, 0)
            gather(1, 1)
            wait_g(0, 0)
            scale(r0, 0)
            gather(2, 2)
            scatter(0, 0)

            def triple(t, c1):
                i = 3 * t
                step(i + 1, 1)
                step(i + 2, 2)
                step(i + 3, 0)
                return c1

            lax.fori_loop(0, (GC - 1) // 3, triple, 0)
            wait_s(GC - 1, (GC - 1) % 3)
            return carry

        lax.fori_loop(0, NG, group, 0)
        plsc.subcore_barrier()

        pltpu.sync_copy(agg_sh.at[pl.ds(sid * RPT, RPT)],
                        out_hbm.at[cid, pl.ds(sid * RPT, RPT)])

    return pl.kernel(
        body,
        out_type=jax.ShapeDtypeStruct((NC, N2, DD), jnp.float32),
        mesh=_mesh,
        compiler_params=pltpu.CompilerParams(use_tc_tiling_on_sc=tc_tiling),
        scratch_types=[
            pltpu.VMEM((GC, CH), jnp.int32),
            pltpu.VMEM((GC, CH), jnp.int32),
            pltpu.VMEM((GC, CH), jnp.float32),
            pltpu.VMEM((CH, DD), jnp.float32),
            pltpu.VMEM((CH, DD), jnp.float32),
            pltpu.VMEM((CH, DD), jnp.float32),
            pltpu.VMEM_SHARED((N2, DD), jnp.float32),
            pltpu.SemaphoreType.DMA,
            pltpu.SemaphoreType.DMA,
            pltpu.SemaphoreType.DMA,
            pltpu.SemaphoreType.DMA,
            pltpu.SemaphoreType.DMA,
            pltpu.SemaphoreType.DMA,
        ],
    )


_sc_layer = _make_sc_layer(D)


def _make_sc_layer_v2(DD, tc_tiling=True, packed=False):
    GD = DD // 2 if packed else DD
    gdt = jnp.int32 if packed else jnp.float32

    def body(m_hbm, src_hbm, dst_hbm, attr_hbm, out_hbm,
             src_v, dst_v, attr_v, b0, b1, b2, f0, f1, agg_sh,
             g0, g1, g2, s0, s1):
        cid = lax.axis_index("c")
        sid = lax.axis_index("s")
        wid = sid * NC + cid
        B = (b0, b1, b2)
        G = (g0, g1, g2)
        F = (f0, f1)
        S = (s0, s1)
        zref = f0

        zv = jnp.zeros((VL,), jnp.float32)

        def zb(i, carry):
            r = i // (DD // VL)
            c = (i % (DD // VL)) * VL
            zref[r, pl.ds(c, VL)] = zv
            return carry

        lax.fori_loop(0, CH * (DD // VL), zb, 0)

        def zslab(t, carry):
            pltpu.sync_copy(zref, agg_sh.at[pl.ds(sid * RPT + t * CH, CH)])
            return carry

        lax.fori_loop(0, RPT // CH, zslab, 0)
        plsc.subcore_barrier()

        def gather(i, b):
            pltpu.async_copy(m_hbm.at[src_v.at[i]], B[b], G[b])

        def wait_g(i, b):
            pltpu.make_async_copy(m_hbm.at[src_v.at[i]], B[b], G[b]).wait()

        def scatter(i, f):
            pltpu.async_copy(F[f], agg_sh.at[dst_v.at[i]], S[f], add=True)

        def wait_s(i, f):
            pltpu.make_async_copy(F[f], agg_sh.at[dst_v.at[i]], S[f]).wait()

        def scale(b, f, i):
            rows_v = B[b]
            out_v = F[f]

            def vgrp(v, c2):
                base = v * VL
                a16 = attr_v[i, pl.ds(base, VL)]
                for jj in range(VL):
                    a = a16[jj]
                    e = base + jj
                    if packed:
                        for k in range(DD // 32):
                            w = rows_v[e, pl.ds(k * VL, VL)]
                            lo = lax.bitcast_convert_type(w * 65536,
                                                          jnp.float32)
                            hi = lax.bitcast_convert_type(w & -65536,
                                                          jnp.float32)
                            out_v[e, pl.ds(k * 2 * VL, VL)] = lo * a
                            out_v[e, pl.ds(k * 2 * VL + VL, VL)] = hi * a
                    else:
                        for k in range(DD // VL):
                            sl = pl.ds(k * VL, VL)
                            out_v[e, sl] = rows_v[e, sl] * a
                return c2

            lax.fori_loop(0, CH // VL, vgrp, 0)

        def step(i, b, f, first=False, pf=True):
            if pf:
                gather(i + 2, (b + 2) % 3)
            wait_g(i, b)
            if not first:
                wait_s(i - 2, f)
            scale(b, f, i)
            scatter(i, f)

        def group(g, carry):
            pltpu.sync_copy(src_hbm.at[wid, g], src_v)
            pltpu.sync_copy(dst_hbm.at[wid, g], dst_v)
            pltpu.sync_copy(attr_hbm.at[wid, g], attr_v)

            gather(0, 0)
            gather(1, 1)
            step(0, 0, 0, first=True)
            step(1, 1, 1, first=True)

            def six(t, c1):
                i = 6 * t + 2
                for k in range(6):
                    step(i + k, (2 + k) % 3, k % 2)
                return c1

            nsix = (GC - 2) // 6
            lax.fori_loop(0, nsix, six, 0)
            for i in range(2 + 6 * nsix, GC):
                step(i, i % 3, i % 2, pf=(i + 2 < GC))
            wait_s(GC - 2, (GC - 2) % 2)
            wait_s(GC - 1, (GC - 1) % 2)
            return carry

        lax.fori_loop(0, NG, group, 0)
        plsc.subcore_barrier()

        pltpu.sync_copy(agg_sh.at[pl.ds(sid * RPT, RPT)],
                        out_hbm.at[cid, pl.ds(sid * RPT, RPT)])

    scratch = [
        pltpu.VMEM((GC, CH), jnp.int32),
        pltpu.VMEM((GC, CH), jnp.int32),
        pltpu.VMEM((GC, CH), jnp.float32),
        pltpu.VMEM((CH, GD), gdt),
        pltpu.VMEM((CH, GD), gdt),
        pltpu.VMEM((CH, GD), gdt),
        pltpu.VMEM((CH, DD), jnp.float32),
        pltpu.VMEM((CH, DD), jnp.float32),
        pltpu.VMEM_SHARED((N2, DD), jnp.float32),
        pltpu.SemaphoreType.DMA,
        pltpu.SemaphoreType.DMA,
        pltpu.SemaphoreType.DMA,
        pltpu.SemaphoreType.DMA,
        pltpu.SemaphoreType.DMA,
    ]
    return pl.kernel(
        body,
        out_type=jax.ShapeDtypeStruct((NC, N2, DD), jnp.float32),
        mesh=_mesh,
        compiler_params=pltpu.CompilerParams(use_tc_tiling_on_sc=tc_tiling),
        scratch_types=scratch,
    )


_sc_layer16 = _make_sc_layer_v2(VL, tc_tiling=False)


def _gru_gates(gi, gh, h):
    i_r, i_z, i_n = gi[:, :D], gi[:, D:2 * D], gi[:, 2 * D:]
    h_r, h_z, h_n = gh[:, :D], gh[:, D:2 * D], gh[:, 2 * D:]
    r = jax.nn.sigmoid(i_r + h_r)
    z = jax.nn.sigmoid(i_z + h_z)
    n = jnp.tanh(i_n + r * h_n)
    return (1.0 - z) * n + z * h


def _gru_math(aggpair, h, wih, whh, bih, bhh):
    agg = aggpair[0] + aggpair[1]
    gi = jnp.dot(agg, wih, preferred_element_type=jnp.float32) + bih
    gh = jnp.dot(h, whh, preferred_element_type=jnp.float32) + bhh
    return _gru_gates(gi, gh, h)


def _gru0_body(s_ref, x_ref, w0r, wih, bih, bhh, whh0, wn, h_out, m_out):
    sp = s_ref[...]
    s = (sp[0] + sp[1])[:, :1]
    u = jnp.dot(w0r[...], wih[...], preferred_element_type=jnp.float32)
    gi = s * u + bih[...]
    x = x_ref[...]
    gh = x * whh0[...] + bhh[...]
    h0 = jnp.concatenate([x, jnp.zeros((x.shape[0], D - 1), jnp.float32)],
                         axis=1)
    hn = _gru_gates(gi, gh, h0)
    h_out[...] = hn
    m_out[...] = jnp.dot(hn, wn[...], preferred_element_type=jnp.float32)


def _gru_body(agg_ref, h_ref, wih, whh, bih, bhh, wn, h_out, m_out):
    hn = _gru_math(agg_ref[...], h_ref[...], wih[...], whh[...], bih[...],
                   bhh[...])
    h_out[...] = hn
    m_out[...] = jnp.dot(hn, wn[...], preferred_element_type=jnp.float32)


def _gru_fin_body(agg_ref, h_ref, wih, whh, bih, bhh, lw, lb, p_out):
    hn = _gru_math(agg_ref[...], h_ref[...], wih[...], whh[...], bih[...],
                   bhh[...])
    logits = jnp.dot(hn, lw[...], preferred_element_type=jnp.float32) + lb[...]
    e = jnp.exp(logits - jnp.max(logits, axis=-1, keepdims=True))
    p_out[...] = e / jnp.sum(e, axis=-1, keepdims=True)


_rows = pl.BlockSpec((BN, D), lambda i: (i, 0))
_aggp = pl.BlockSpec((2, BN, D), lambda i: (0, i, 0))
_full = lambda shape: pl.BlockSpec(shape, lambda i: tuple(0 for _ in shape))
_grid = (N // BN,)

_gru0 = pl.pallas_call(
    _gru0_body,
    grid=_grid,
    in_specs=[pl.BlockSpec((2, BN, VL), lambda i: (0, i, 0)),
              pl.BlockSpec((BN, 1), lambda i: (i, 0)),
              _full((1, D)), _full((D, 3 * D)), _full((1, 3 * D)),
              _full((1, 3 * D)), _full((1, 3 * D)), _full((D, D))],
    out_specs=[_rows, _rows],
    out_shape=[jax.ShapeDtypeStruct((N, D), jnp.float32),
               jax.ShapeDtypeStruct((N, D), jnp.float32)],
)

_gru = pl.pallas_call(
    _gru_body,
    grid=_grid,
    in_specs=[_aggp, _rows,
              _full((D, 3 * D)), _full((D, 3 * D)), _full((1, 3 * D)),
              _full((1, 3 * D)), _full((D, D))],
    out_specs=[_rows, _rows],
    out_shape=[jax.ShapeDtypeStruct((N, D), jnp.float32),
               jax.ShapeDtypeStruct((N, D), jnp.float32)],
)

_gru_fin = pl.pallas_call(
    _gru_fin_body,
    grid=_grid,
    in_specs=[_aggp, _rows,
              _full((D, 3 * D)), _full((D, 3 * D)), _full((1, 3 * D)),
              _full((1, 3 * D)), _full((D, K)), _full((1, K))],
    out_specs=pl.BlockSpec((BN, K), lambda i: (i, 0)),
    out_shape=jax.ShapeDtypeStruct((N, K), jnp.float32),
)


@jax.jit
def kernel(x, edge_index, edge_attr, batch, weight, w_ih, w_hh, b_ih, b_hh,
           lin_w, lin_b):
    src3 = edge_index[0].reshape(NW, NG, GC, CH)
    dst3 = edge_index[1].reshape(NW, NG, GC, CH)
    attr3 = edge_attr.reshape(NW, NG, GC, CH)
    wihT = w_ih.T
    whhT = w_hh.T
    bih = b_ih.reshape(1, 3 * D)
    bhh = b_hh.reshape(1, 3 * D)
    w0row = weight[0, 0].reshape(1, D)
    whh0 = w_hh[:, 0].reshape(1, 3 * D)
    lwT = lin_w.T
    lb = lin_b.reshape(1, K)
    x16 = jnp.broadcast_to(x, (N, VL))

    s2 = _sc_layer16(x16, src3, dst3, attr3)
    h, m = _gru0(s2, x, w0row, wihT, bih, bhh, whh0, weight[1])
    aggs = _sc_layer(m, src3, dst3, attr3)
    h, m = _gru(aggs, h, wihT, whhT, bih, bhh, weight[2])
    aggs = _sc_layer(m, src3, dst3, attr3)
    h, m = _gru(aggs, h, wihT, whhT, bih, bhh, weight[3])
    aggs = _sc_layer(m, src3, dst3, attr3)
    return _gru_fin(aggs, h, wihT, whhT, bih, bhh, lwT, lb)


# speedup vs baseline: 1.0021x; 1.0021x over previous
import jax
import jax.numpy as jnp
from jax import lax
from jax.experimental import pallas as pl
from jax.experimental.pallas import tpu as pltpu
from jax.experimental.pallas import tpu_sc as plsc

N = 10000
E = 320000
D = 128
K = 2
NC = 2
NS = 16
NW = NC * NS
EW = E // NW
CH = 80
NCHUNK = EW // CH
NG = 5
GC = NCHUNK // NG
N2 = 10240
RPT = N2 // NS
ZR = 32
VL = 16
BN = 2000

_mesh = plsc.VectorSubcoreMesh(core_axis_name="c", subcore_axis_name="s")


def _make_sc_layer(DD, tc_tiling=True):
    def body(m_hbm, src_hbm, dst_hbm, attr_hbm, out_hbm,
             src_v, dst_v, attr_v, r0, r1, r2, agg_sh,
             g0, g1, g2, s0, s1, s2):
        cid = lax.axis_index("c")
        sid = lax.axis_index("s")
        wid = sid * NC + cid
        R = (r0, r1, r2)
        G = (g0, g1, g2)
        S = (s0, s1, s2)

        zv = jnp.zeros((VL,), jnp.float32)

        def zb(i, carry):
            r = i // (DD // VL)
            c = (i % (DD // VL)) * VL
            r0[r, pl.ds(c, VL)] = zv
            return carry

        lax.fori_loop(0, CH * (DD // VL), zb, 0)

        def zslab(t, carry):
            pltpu.sync_copy(r0, agg_sh.at[pl.ds(sid * RPT + t * CH, CH)])
            return carry

        lax.fori_loop(0, RPT // CH, zslab, 0)
        plsc.subcore_barrier()

        def gather(i, b):
            pltpu.async_copy(m_hbm.at[src_v.at[i]], R[b], G[b])

        def wait_g(i, b):
            pltpu.make_async_copy(m_hbm.at[src_v.at[i]], R[b], G[b]).wait()

        def scatter(i, b):
            pltpu.async_copy(R[b], agg_sh.at[dst_v.at[i]], S[b], add=True)

        def wait_s(i, b):
            pltpu.make_async_copy(R[b], agg_sh.at[dst_v.at[i]], S[b]).wait()

        def scale(rows_v, i):
            def vgrp(v, c2):
                base = v * VL
                a16 = attr_v[i, pl.ds(base, VL)]
                for jj in range(VL):
                    a = a16[jj]
                    for k in range(DD // VL):
                        sl = pl.ds(k * VL, VL)
                        rows_v[base + jj, sl] = rows_v[base + jj, sl] * a
                return c2

            lax.fori_loop(0, CH // VL, vgrp, 0)

        def step(i, b):
            pb = (b + 2) % 3
            wait_g(i, b)
            scale(R[b], i)
            wait_s(i, pb)

            @pl.when(i + 2 < GC)
            def _():
                gather(i + 2, pb)

            scatter(i, b)

        def group(g, carry):
            pltpu.sync_copy(src_hbm.at[wid, g], src_v)
            pltpu.sync_copy(dst_hbm.at[wid, g], dst_v)
            pltpu.sync_copy(attr_hbm.at[wid, g], attr_v)

            gather(0, 0)
            gather(1, 1)
            wait_g(0, 0)
            scale(r0, 0)
            gather(2, 2)
            scatter(0, 0)

            def triple(t, c1):
                i = 3 * t
                step(i + 1, 1)
                step(i + 2, 2)
                step(i + 3, 0)
                return c1

            lax.fori_loop(0, (GC - 1) // 3, triple, 0)
            wait_s(GC - 1, (GC - 1) % 3)
            return carry

        lax.fori_loop(0, NG, group, 0)
        plsc.subcore_barrier()

        pltpu.sync_copy(agg_sh.at[pl.ds(sid * RPT, RPT)],
                        out_hbm.at[cid, pl.ds(sid * RPT, RPT)])

    return pl.kernel(
        body,
        out_type=jax.ShapeDtypeStruct((NC, N2, DD), jnp.float32),
        mesh=_mesh,
        compiler_params=pltpu.CompilerParams(use_tc_tiling_on_sc=tc_tiling),
        scratch_types=[
            pltpu.VMEM((GC, CH), jnp.int32),
            pltpu.VMEM((GC, CH), jnp.int32),
            pltpu.VMEM((GC, CH), jnp.float32),
            pltpu.VMEM((CH, DD), jnp.float32),
            pltpu.VMEM((CH, DD), jnp.float32),
            pltpu.VMEM((CH, DD), jnp.float32),
            pltpu.VMEM_SHARED((N2, DD), jnp.float32),
            pltpu.SemaphoreType.DMA,
            pltpu.SemaphoreType.DMA,
            pltpu.SemaphoreType.DMA,
            pltpu.SemaphoreType.DMA,
            pltpu.SemaphoreType.DMA,
            pltpu.SemaphoreType.DMA,
        ],
    )


_sc_layer = _make_sc_layer(D)


def _make_sc_layer_v2(DD, tc_tiling=True):
    GD = DD
    gdt = jnp.float32

    def body(m_hbm, src_hbm, dst_hbm, attr_hbm, out_hbm,
             src_v, dst_v, attr_v, b0, b1, b2, f0, f1, agg_sh,
             g0, g1, g2, s0, s1):
        cid = lax.axis_index("c")
        sid = lax.axis_index("s")
        wid = sid * NC + cid
        B = (b0, b1, b2)
        G = (g0, g1, g2)
        F = (f0, f1)
        S = (s0, s1)
        zref = f0

        zv = jnp.zeros((VL,), jnp.float32)

        def zb(i, carry):
            r = i // (DD // VL)
            c = (i % (DD // VL)) * VL
            zref[r, pl.ds(c, VL)] = zv
            return carry

        lax.fori_loop(0, CH * (DD // VL), zb, 0)

        def zslab(t, carry):
            pltpu.sync_copy(zref, agg_sh.at[pl.ds(sid * RPT + t * CH, CH)])
            return carry

        lax.fori_loop(0, RPT // CH, zslab, 0)
        plsc.subcore_barrier()

        def gather(i, b):
            pltpu.async_copy(m_hbm.at[src_v.at[i]], B[b], G[b])

        def wait_g(i, b):
            pltpu.make_async_copy(m_hbm.at[src_v.at[i]], B[b], G[b]).wait()

        def scatter(i, f):
            pltpu.async_copy(F[f], agg_sh.at[dst_v.at[i]], S[f], add=True)

        def wait_s(i, f):
            pltpu.make_async_copy(F[f], agg_sh.at[dst_v.at[i]], S[f]).wait()

        def scale(b, f, i):
            rows_v = B[b]
            out_v = F[f]

            def vgrp(v, c2):
                base = v * VL
                a16 = attr_v[i, pl.ds(base, VL)]
                for jj in range(VL):
                    a = a16[jj]
                    e = base + jj
                    for k in range(DD // VL):
                        sl = pl.ds(k * VL, VL)
                        out_v[e, sl] = rows_v[e, sl] * a
                return c2

            lax.fori_loop(0, CH // VL, vgrp, 0)

        def step(i, b, f, first=False, pf=True):
            if pf:
                gather(i + 2, (b + 2) % 3)
            wait_g(i, b)
            if not first:
                wait_s(i - 2, f)
            scale(b, f, i)
            scatter(i, f)

        def group(g, carry):
            pltpu.sync_copy(src_hbm.at[wid, g], src_v)
            pltpu.sync_copy(dst_hbm.at[wid, g], dst_v)
            pltpu.sync_copy(attr_hbm.at[wid, g], attr_v)

            gather(0, 0)
            gather(1, 1)
            step(0, 0, 0, first=True)
            step(1, 1, 1, first=True)

            def six(t, c1):
                i = 6 * t + 2
                for k in range(6):
                    step(i + k, (2 + k) % 3, k % 2)
                return c1

            nsix = (GC - 2) // 6
            lax.fori_loop(0, nsix, six, 0)
            for i in range(2 + 6 * nsix, GC):
                step(i, i % 3, i % 2, pf=(i + 2 < GC))
            wait_s(GC - 2, (GC - 2) % 2)
            wait_s(GC - 1, (GC - 1) % 2)
            return carry

        lax.fori_loop(0, NG, group, 0)
        plsc.subcore_barrier()

        pltpu.sync_copy(agg_sh.at[pl.ds(sid * RPT, RPT)],
                        out_hbm.at[cid, pl.ds(sid * RPT, RPT)])

    scratch = [
        pltpu.VMEM((GC, CH), jnp.int32),
        pltpu.VMEM((GC, CH), jnp.int32),
        pltpu.VMEM((GC, CH), jnp.float32),
        pltpu.VMEM((CH, GD), gdt),
        pltpu.VMEM((CH, GD), gdt),
        pltpu.VMEM((CH, GD), gdt),
        pltpu.VMEM((CH, DD), jnp.float32),
        pltpu.VMEM((CH, DD), jnp.float32),
        pltpu.VMEM_SHARED((N2, DD), jnp.float32),
        pltpu.SemaphoreType.DMA,
        pltpu.SemaphoreType.DMA,
        pltpu.SemaphoreType.DMA,
        pltpu.SemaphoreType.DMA,
        pltpu.SemaphoreType.DMA,
    ]
    return pl.kernel(
        body,
        out_type=jax.ShapeDtypeStruct((NC, N2, DD), jnp.float32),
        mesh=_mesh,
        compiler_params=pltpu.CompilerParams(use_tc_tiling_on_sc=tc_tiling),
        scratch_types=scratch,
    )


_sc_layer16 = _make_sc_layer_v2(VL, tc_tiling=False)


def _gru_gates(gi, gh, h):
    i_r, i_z, i_n = gi[:, :D], gi[:, D:2 * D], gi[:, 2 * D:]
    h_r, h_z, h_n = gh[:, :D], gh[:, D:2 * D], gh[:, 2 * D:]
    r = jax.nn.sigmoid(i_r + h_r)
    z = jax.nn.sigmoid(i_z + h_z)
    n = jnp.tanh(i_n + r * h_n)
    return (1.0 - z) * n + z * h


def _gru_math(aggpair, h, wih, whh, bih, bhh):
    agg = aggpair[0] + aggpair[1]
    gi = jnp.dot(agg, wih, preferred_element_type=jnp.float32) + bih
    gh = jnp.dot(h, whh, preferred_element_type=jnp.float32) + bhh
    return _gru_gates(gi, gh, h)


def _gru0_body(s_ref, x_ref, w0r, wih, bih, bhh, whh0, wn, h_out, m_out):
    sp = s_ref[...]
    s = (sp[0] + sp[1])[:, :1]
    u = jnp.dot(w0r[...], wih[...], preferred_element_type=jnp.float32)
    gi = s * u + bih[...]
    x = x_ref[...]
    gh = x * whh0[...] + bhh[...]
    h0 = jnp.concatenate([x, jnp.zeros((x.shape[0], D - 1), jnp.float32)],
                         axis=1)
    hn = _gru_gates(gi, gh, h0)
    h_out[...] = hn
    m_out[...] = jnp.dot(hn, wn[...], preferred_element_type=jnp.float32)


def _gru_body(agg_ref, h_ref, wih, whh, bih, bhh, wn, h_out, m_out):
    hn = _gru_math(agg_ref[...], h_ref[...], wih[...], whh[...], bih[...],
                   bhh[...])
    h_out[...] = hn
    m_out[...] = jnp.dot(hn, wn[...], preferred_element_type=jnp.float32)


def _gru_fin_body(agg_ref, h_ref, wih, whh, bih, bhh, lw, lb, p_out):
    hn = _gru_math(agg_ref[...], h_ref[...], wih[...], whh[...], bih[...],
                   bhh[...])
    logits = jnp.dot(hn, lw[...], preferred_element_type=jnp.float32) + lb[...]
    e = jnp.exp(logits - jnp.max(logits, axis=-1, keepdims=True))
    p_out[...] = e / jnp.sum(e, axis=-1, keepdims=True)


_rows = pl.BlockSpec((BN, D), lambda i: (i, 0))
_aggp = pl.BlockSpec((2, BN, D), lambda i: (0, i, 0))
_full = lambda shape: pl.BlockSpec(shape, lambda i: tuple(0 for _ in shape))
_grid = (N // BN,)

_gru0 = pl.pallas_call(
    _gru0_body,
    grid=_grid,
    in_specs=[pl.BlockSpec((2, BN, VL), lambda i: (0, i, 0)),
              pl.BlockSpec((BN, 1), lambda i: (i, 0)),
              _full((1, D)), _full((D, 3 * D)), _full((1, 3 * D)),
              _full((1, 3 * D)), _full((1, 3 * D)), _full((D, D))],
    out_specs=[_rows, _rows],
    out_shape=[jax.ShapeDtypeStruct((N, D), jnp.float32),
               jax.ShapeDtypeStruct((N, D), jnp.float32)],
)

_gru = pl.pallas_call(
    _gru_body,
    grid=_grid,
    in_specs=[_aggp, _rows,
              _full((D, 3 * D)), _full((D, 3 * D)), _full((1, 3 * D)),
              _full((1, 3 * D)), _full((D, D))],
    out_specs=[_rows, _rows],
    out_shape=[jax.ShapeDtypeStruct((N, D), jnp.float32),
               jax.ShapeDtypeStruct((N, D), jnp.float32)],
)

_gru_fin = pl.pallas_call(
    _gru_fin_body,
    grid=_grid,
    in_specs=[_aggp, _rows,
              _full((D, 3 * D)), _full((D, 3 * D)), _full((1, 3 * D)),
              _full((1, 3 * D)), _full((D, K)), _full((1, K))],
    out_specs=pl.BlockSpec((BN, K), lambda i: (i, 0)),
    out_shape=jax.ShapeDtypeStruct((N, K), jnp.float32),
)


@jax.jit
def kernel(x, edge_index, edge_attr, batch, weight, w_ih, w_hh, b_ih, b_hh,
           lin_w, lin_b):
    src3 = edge_index[0].reshape(NW, NG, GC, CH)
    dst3 = edge_index[1].reshape(NW, NG, GC, CH)
    attr3 = edge_attr.reshape(NW, NG, GC, CH)
    wihT = w_ih.T
    whhT = w_hh.T
    bih = b_ih.reshape(1, 3 * D)
    bhh = b_hh.reshape(1, 3 * D)
    w0row = weight[0, 0].reshape(1, D)
    whh0 = w_hh[:, 0].reshape(1, 3 * D)
    lwT = lin_w.T
    lb = lin_b.reshape(1, K)
    x16 = jnp.broadcast_to(x, (N, VL))

    s2 = _sc_layer16(x16, src3, dst3, attr3)
    h, m = _gru0(s2, x, w0row, wihT, bih, bhh, whh0, weight[1])
    aggs = _sc_layer(m, src3, dst3, attr3)
    h, m = _gru(aggs, h, wihT, whhT, bih, bhh, weight[2])
    aggs = _sc_layer(m, src3, dst3, attr3)
    h, m = _gru(aggs, h, wihT, whhT, bih, bhh, weight[3])
    aggs = _sc_layer(m, src3, dst3, attr3)
    return _gru_fin(aggs, h, wihT, whhT, bih, bhh, lwT, lb)
